# baseline (speedup 1.0000x reference)
import jax
import jax.numpy as jnp
from jax import lax
from jax.experimental import pallas as pl
from jax.experimental.pallas import tpu as pltpu
from jax.experimental.pallas import tpu_sc as plsc

C = 128
LANES = 16
NSUB = 16
NTOK = 16384
TPW = NTOK // NSUB
RPW = 8
COLS = TPW // RPW
VPR = COLS // LANES
NBIN = LANES * C
DEAD = NBIN
MROWS = NTOK // COLS
NCHUNK = NTOK * C // LANES


def _wce_body(preds_hbm, ml_hbm, out_hbm,
              ml_v, idx_v, bidx_v, off_v, g_v, cntb, psumb, part_v,
              allp_v, out_v, shared, sem, sem2):
    w = lax.axis_index("s")
    base = w * TPW
    pltpu.async_copy(ml_hbm.at[pl.ds(w * RPW, RPW)], ml_v, sem2).wait()

    iota = lax.iota(jnp.int32, LANES)
    lane_row = iota * C
    dead = DEAD + iota
    zerov = jnp.zeros((LANES,), jnp.float32)
    onev = jnp.ones((LANES,), jnp.float32)

    gcopies = []
    with jax.named_scope("p1_idx"):
        for r in range(RPW):
            rbase = (base + r * COLS) * C
            for k in range(VPR):
                sl = pl.ds(k * LANES, LANES)
                ml = ml_v[r, sl]
                live = lane_row + (ml & (C - 1))
                e = rbase + k * (LANES * C) + live
                bidx_v[r, sl] = jnp.where(ml >= 256, live, dead)
                idx_v[r, sl] = lax.shift_right_logical(e, 4)
                off_v[r, sl] = e & (LANES - 1)
            gcopies.append(pltpu.async_copy(
                preds_hbm.at[idx_v.at[r]], g_v.at[r], sem))

    with jax.named_scope("p2_zero"):
        for i in range(NBIN // LANES):
            cntb[pl.ds(i * LANES, LANES)] = zerov
            psumb[pl.ds(i * LANES, LANES)] = zerov

    with jax.named_scope("p3_cnt"):
        for r in range(RPW):
            for k in range(VPR):
                plsc.addupdate_scatter(
                    cntb, [bidx_v[r, pl.ds(k * LANES, LANES)]], onev)
        for k in range(C // LANES):
            acs = [cntb[pl.ds(l * C + k * LANES, LANES)] for l in range(LANES)]
            while len(acs) > 1:
                acs = [acs[i] + acs[i + 1] for i in range(0, len(acs), 2)]
            part_v[pl.ds(k * LANES, LANES)] = acs[0]

    with jax.named_scope("p4_psum"):
        for r in range(RPW):
            gcopies[r].wait()
            for k in range(VPR):
                sl = pl.ds(k * LANES, LANES)
                g = plsc.load_gather(
                    g_v.at[r], [k * LANES + iota, off_v[r, sl]])
                plsc.addupdate_scatter(psumb, [bidx_v[r, sl]], g)
        for k in range(C // LANES):
            aps = [psumb[pl.ds(l * C + k * LANES, LANES)] for l in range(LANES)]
            while len(aps) > 1:
                aps = [aps[i] + aps[i + 1] for i in range(0, len(aps), 2)]
            part_v[pl.ds(C + k * LANES, LANES)] = aps[0]

    with jax.named_scope("p5_pub"):
        pltpu.sync_copy(part_v, shared.at[w])
        plsc.subcore_barrier()

    @pl.when(w == 0)
    def _final():
      with jax.named_scope("p6_final"):
        pltpu.sync_copy(shared, allp_v)
        cnt, ps = [], []
        for k in range(C // LANES):
            acs = [allp_v[t, pl.ds(k * LANES, LANES)] for t in range(NSUB)]
            aps = [allp_v[t, pl.ds(C + k * LANES, LANES)] for t in range(NSUB)]
            while len(acs) > 1:
                acs = [acs[i] + acs[i + 1] for i in range(0, len(acs), 2)]
                aps = [aps[i] + aps[i + 1] for i in range(0, len(aps), 2)]
            cnt.append(acs[0])
            ps.append(aps[0])
        mv = cnt[0]
        for k in range(1, C // LANES):
            mv = jnp.minimum(mv, cnt[k])
        mmin = jnp.min(mv)
        num = jnp.zeros((LANES,), jnp.float32)
        den = jnp.zeros((LANES,), jnp.float32)
        for k in range(C // LANES):
            wgt = mmin / (cnt[k] + 1e-8)
            num = num + wgt * ps[k]
            den = den + wgt * cnt[k]
        numv = jnp.full((LANES,), jnp.sum(num), jnp.float32)
        denv = jnp.full((LANES,), jnp.sum(den), jnp.float32)
        out_v[...] = -(numv / denv)
        pltpu.sync_copy(out_v, out_hbm)


def kernel(preds, labels, pad_mask):
    b, s, c = preds.shape
    preds_ch = preds.reshape(NCHUNK, LANES)
    ml = (labels.astype(jnp.int32)
          | (pad_mask.astype(jnp.int32) << 8)).reshape(MROWS, COLS)
    mesh = plsc.VectorSubcoreMesh(
        core_axis_name="c", subcore_axis_name="s", num_cores=1)
    out = pl.kernel(
        _wce_body,
        out_type=jax.ShapeDtypeStruct((LANES,), jnp.float32),
        mesh=mesh,
        compiler_params=pltpu.CompilerParams(
            needs_layout_passes=False, use_tc_tiling_on_sc=False),
        scratch_types=[
            pltpu.VMEM((RPW, COLS), jnp.int32),
            pltpu.VMEM((RPW, COLS), jnp.int32),
            pltpu.VMEM((RPW, COLS), jnp.int32),
            pltpu.VMEM((RPW, COLS), jnp.int32),
            pltpu.VMEM((RPW, COLS, LANES), jnp.float32),
            pltpu.VMEM((NBIN + LANES,), jnp.float32),
            pltpu.VMEM((NBIN + LANES,), jnp.float32),
            pltpu.VMEM((2 * C,), jnp.float32),
            pltpu.VMEM((NSUB, 2 * C), jnp.float32),
            pltpu.VMEM((LANES,), jnp.float32),
            pltpu.VMEM_SHARED((NSUB, 2 * C), jnp.float32),
            pltpu.SemaphoreType.DMA,
            pltpu.SemaphoreType.DMA,
        ],
    )(preds_ch, ml)
    return out[0]


# speedup vs baseline: 1.0264x; 1.0264x over previous
import jax
import jax.numpy as jnp
from jax import lax
from jax.experimental import pallas as pl
from jax.experimental.pallas import tpu as pltpu
from jax.experimental.pallas import tpu_sc as plsc

C = 128
LANES = 16
NSUB = 16
NTOK = 16384
TPW = NTOK // NSUB
RPW = 8
COLS = TPW // RPW
VPR = COLS // LANES
NBIN = LANES * C
DEAD = NBIN
MROWS = NTOK // COLS


def _wce_body(preds_hbm, ml_hbm, out_hbm,
              ml_v, idx_v, bidx_v, g_v, cntb, psumb, part_v,
              allp_v, out_v, shared, sem, sem2):
    w = lax.axis_index("s")
    base = w * TPW
    pltpu.async_copy(ml_hbm.at[pl.ds(w * RPW, RPW)], ml_v, sem2).wait()

    iota = lax.iota(jnp.int32, LANES)
    lane_row = iota * C
    dead = DEAD + iota
    zerov = jnp.zeros((LANES,), jnp.float32)
    onev = jnp.ones((LANES,), jnp.float32)

    gcopies = []
    for r in range(RPW):
        rbase = (base + r * COLS) * C
        for k in range(VPR):
            sl = pl.ds(k * LANES, LANES)
            ml = ml_v[r, sl]
            live = lane_row + (ml & (C - 1))
            bidx_v[r, sl] = jnp.where(ml >= 256, live, dead)
            idx_v[r, sl] = rbase + k * (LANES * C) + live
        gcopies.append(pltpu.async_copy(preds_hbm.at[idx_v.at[r]], g_v.at[r], sem))

    for i in range(NBIN // LANES):
        cntb[pl.ds(i * LANES, LANES)] = zerov
        psumb[pl.ds(i * LANES, LANES)] = zerov

    for r in range(RPW):
        for k in range(VPR):
            plsc.addupdate_scatter(
                cntb, [bidx_v[r, pl.ds(k * LANES, LANES)]], onev)
    for k in range(C // LANES):
        sl = pl.ds(k * LANES, LANES)
        ac = cntb[sl]
        for l in range(1, LANES):
            ac = ac + cntb[pl.ds(l * C + k * LANES, LANES)]
        part_v[sl] = ac

    for r in range(RPW):
        gcopies[r].wait()
        for k in range(VPR):
            sl = pl.ds(k * LANES, LANES)
            plsc.addupdate_scatter(psumb, [bidx_v[r, sl]], g_v[r, sl])
    for k in range(C // LANES):
        sl = pl.ds(k * LANES, LANES)
        ap = psumb[sl]
        for l in range(1, LANES):
            ap = ap + psumb[pl.ds(l * C + k * LANES, LANES)]
        part_v[pl.ds(C + k * LANES, LANES)] = ap

    pltpu.sync_copy(part_v, shared.at[w])
    plsc.subcore_barrier()

    @pl.when(w == 0)
    def _final():
        pltpu.sync_copy(shared, allp_v)
        cnt, ps = [], []
        for k in range(C // LANES):
            ac = allp_v[0, pl.ds(k * LANES, LANES)]
            ap = allp_v[0, pl.ds(C + k * LANES, LANES)]
            for t in range(1, NSUB):
                ac = ac + allp_v[t, pl.ds(k * LANES, LANES)]
                ap = ap + allp_v[t, pl.ds(C + k * LANES, LANES)]
            cnt.append(ac)
            ps.append(ap)
        mv = cnt[0]
        for k in range(1, C // LANES):
            mv = jnp.minimum(mv, cnt[k])
        mmin = jnp.min(mv)
        num = jnp.zeros((LANES,), jnp.float32)
        den = jnp.zeros((LANES,), jnp.float32)
        for k in range(C // LANES):
            wgt = mmin / (cnt[k] + 1e-8)
            num = num + wgt * ps[k]
            den = den + wgt * cnt[k]
        numv = jnp.full((LANES,), jnp.sum(num), jnp.float32)
        denv = jnp.full((LANES,), jnp.sum(den), jnp.float32)
        out_v[...] = -(numv / denv)
        pltpu.sync_copy(out_v, out_hbm)


def kernel(preds, labels, pad_mask):
    b, s, c = preds.shape
    preds_f = preds.reshape(b * s * c)
    ml = (labels.astype(jnp.int32)
          | (pad_mask.astype(jnp.int32) << 8)).reshape(MROWS, COLS)
    mesh = plsc.VectorSubcoreMesh(
        core_axis_name="c", subcore_axis_name="s", num_cores=1)
    out = pl.kernel(
        _wce_body,
        out_type=jax.ShapeDtypeStruct((LANES,), jnp.float32),
        mesh=mesh,
        compiler_params=pltpu.CompilerParams(needs_layout_passes=False),
        scratch_types=[
            pltpu.VMEM((RPW, COLS), jnp.int32),
            pltpu.VMEM((RPW, COLS), jnp.int32),
            pltpu.VMEM((RPW, COLS), jnp.int32),
            pltpu.VMEM((RPW, COLS), jnp.float32),
            pltpu.VMEM((NBIN + LANES,), jnp.float32),
            pltpu.VMEM((NBIN + LANES,), jnp.float32),
            pltpu.VMEM((2 * C,), jnp.float32),
            pltpu.VMEM((NSUB, 2 * C), jnp.float32),
            pltpu.VMEM((LANES,), jnp.float32),
            pltpu.VMEM_SHARED((NSUB, 2 * C), jnp.float32),
            pltpu.SemaphoreType.DMA,
            pltpu.SemaphoreType.DMA,
        ],
    )(preds_f, ml)
    return out[0]
